# baseline (speedup 1.0000x reference)
import functools

import jax
import jax.numpy as jnp
from jax import lax
from jax.experimental import pallas as pl
from jax.experimental.pallas import tpu as pltpu
from jax.experimental.pallas import tpu_sc as plsc

_BATCH = 16384
_DIM = 32
_ROWS = 100000
_MAIN = 99968
_TAIL = _ROWS - _MAIN
_REPS = 2
_ICH = 4096
_PCH = ((0, 25600), (25600, 25600), (51200, 25600), (76800, 23168))
_NICH = _BATCH // _ICH
_UNROLL = 16
_LANES = 16
_TPD = _BATCH // 128


def _build():
  mesh = plsc.VectorSubcoreMesh(core_axis_name="c", subcore_axis_name="s")

  @functools.partial(
      pl.kernel,
      mesh=mesh,
      out_type=jax.ShapeDtypeStruct((2 * _DIM * _BATCH // 128, 128),
                                    jnp.float32),
      scratch_types=[
          pltpu.VMEM((1, _MAIN + 128), jnp.float32),
          pltpu.VMEM((2, _ICH), jnp.int32),
          pltpu.VMEM((_TPD, 128), jnp.float32),
          pltpu.VMEM((2, _TPD), jnp.int32),
          pltpu.VMEM((_LANES,), jnp.int32),
          pltpu.SemaphoreType.DMA,
          pltpu.SemaphoreType.DMA,
          pltpu.SemaphoreType.DMA,
      ],
      compiler_params=pltpu.CompilerParams(needs_layout_passes=False),
  )
  def emb(wf3, wa3, tf3, ta3, fidx, aidx, out,
          row_v, idx_v, out_v, sidx_v, idx1_v, sem_r, sem_i, sem_o):
    c = lax.axis_index("c")
    s = lax.axis_index("s")

    out_cp = None
    for rep in range(_REPS):
      d = s * _REPS + rep
      sdim = d // 8
      r = d % 8
      idx1_v[pl.ds(0, _LANES)] = r + lax.iota(jnp.int32, _LANES) * 0

      @pl.when(c == 0)
      def _():
        pltpu.async_copy(
            tf3.at[sdim].at[idx1_v.at[pl.ds(0, 1)]],
            row_v.at[:, pl.ds(_MAIN, 128)], sem_r)
        for off, sz in _PCH:
          pltpu.async_copy(
              wf3.at[sdim, :, pl.ds(off, sz)].at[idx1_v.at[pl.ds(0, 1)]],
              row_v.at[:, pl.ds(off, sz)], sem_r)

      @pl.when(c == 1)
      def _():
        pltpu.async_copy(
            ta3.at[sdim].at[idx1_v.at[pl.ds(0, 1)]],
            row_v.at[:, pl.ds(_MAIN, 128)], sem_r)
        for off, sz in _PCH:
          pltpu.async_copy(
              wa3.at[sdim, :, pl.ds(off, sz)].at[idx1_v.at[pl.ds(0, 1)]],
              row_v.at[:, pl.ds(off, sz)], sem_r)

      idx_cps = []
      for ch in range(2):
        @pl.when(c == 0)
        def _():
          pltpu.async_copy(
              fidx.at[pl.ds(ch * _ICH, _ICH)], idx_v.at[ch], sem_i)

        @pl.when(c == 1)
        def _():
          pltpu.async_copy(
              aidx.at[pl.ds(ch * _ICH, _ICH)], idx_v.at[ch], sem_i)

      base = (c * 4 + sdim) * (8 * _TPD) + r
      def sbody(k, _):
        t16 = lax.iota(jnp.int32, _LANES) + k * _LANES
        sidx_v[rep, pl.ds(k * _LANES, _LANES)] = base + 8 * t16
        return 0
      lax.fori_loop(0, _TPD // _LANES, sbody, 0)

      for off, sz in _PCH:
        pltpu.make_async_copy(
            wf3.at[0, :, pl.ds(off, sz)].at[idx1_v.at[pl.ds(0, 1)]],
            row_v.at[:, pl.ds(off, sz)], sem_r).wait()
      pltpu.make_async_copy(
          tf3.at[0].at[idx1_v.at[pl.ds(0, 1)]],
          row_v.at[:, pl.ds(_MAIN, 128)], sem_r).wait()

      if out_cp is not None:
        out_cp.wait()

      for ch in range(_NICH):
        buf = ch % 2
        pltpu.make_async_copy(
            fidx.at[pl.ds(0, _ICH)], idx_v.at[buf], sem_i).wait()

        def body(i, _):
          base_i = i * (_UNROLL * _LANES)
          for u in range(_UNROLL):
            off = base_i + u * _LANES
            iv = idx_v[buf, pl.ds(off, _LANES)]
            vals = plsc.load_gather(row_v, [iv * 0, iv])
            orow = (ch * _ICH + off) // 128
            ocol = off % 128
            out_v[orow, pl.ds(ocol, _LANES)] = vals
          return 0

        lax.fori_loop(0, _ICH // (_UNROLL * _LANES), body, 0)

        if ch + 2 < _NICH:
          nxt = ch + 2

          @pl.when(c == 0)
          def _():
            pltpu.async_copy(
                fidx.at[pl.ds(nxt * _ICH, _ICH)], idx_v.at[buf], sem_i)

          @pl.when(c == 1)
          def _():
            pltpu.async_copy(
                aidx.at[pl.ds(nxt * _ICH, _ICH)], idx_v.at[buf], sem_i)

      out_cp = pltpu.async_copy(out_v, out.at[sidx_v.at[rep]], sem_o)
    out_cp.wait()

  return emb


_emb = _build()


def kernel(user_fea, W_fans, W_avgrating):
  fans_idx = user_fea[:, 0].astype(jnp.int32)
  avg_idx = user_fea[:, 1].astype(jnp.int32)
  wf3 = W_fans.T.reshape(4, 8, _ROWS)
  wa3 = W_avgrating.T.reshape(4, 8, _ROWS)
  tf3 = jnp.pad(W_fans.T[:, _MAIN:],
                ((0, 0), (0, 128 - _TAIL))).reshape(4, 8, 128)
  ta3 = jnp.pad(W_avgrating.T[:, _MAIN:],
                ((0, 0), (0, 128 - _TAIL))).reshape(4, 8, 128)
  out4 = _emb(wf3, wa3, tf3, ta3, fans_idx, avg_idx)
  out_t = (out4.reshape(8, 128, 8, 128)
           .transpose(0, 2, 1, 3)
           .reshape(2 * _DIM, _BATCH))
  return out_t.T


# speedup vs baseline: 1.0119x; 1.0119x over previous
import functools

import jax
import jax.numpy as jnp
from jax import lax
from jax.experimental import pallas as pl
from jax.experimental.pallas import tpu as pltpu
from jax.experimental.pallas import tpu_sc as plsc

_BATCH = 16384
_DIM = 32
_ROWS = 100000
_MAIN = 99968
_TAIL = _ROWS - _MAIN
_REPS = 2
_ICH = 4096
_PCH = ((0, 25600), (25600, 25600), (51200, 25600), (76800, 23168))
_NICH = _BATCH // _ICH
_UNROLL = 16
_LANES = 16
_TPD = _BATCH // 128


def _build():
  mesh = plsc.VectorSubcoreMesh(core_axis_name="c", subcore_axis_name="s")

  @functools.partial(
      pl.kernel,
      mesh=mesh,
      out_type=jax.ShapeDtypeStruct((2 * _DIM * _BATCH // 128, 128),
                                    jnp.float32),
      scratch_types=[
          pltpu.VMEM((1, _MAIN + 128), jnp.float32),
          pltpu.VMEM((2, _ICH), jnp.int32),
          pltpu.VMEM((_TPD, 128), jnp.float32),
          pltpu.VMEM((2, _TPD), jnp.int32),
          pltpu.VMEM((_LANES,), jnp.int32),
          pltpu.SemaphoreType.DMA,
          pltpu.SemaphoreType.DMA,
          pltpu.SemaphoreType.DMA,
      ],
      compiler_params=pltpu.CompilerParams(needs_layout_passes=False),
  )
  def emb(wf3, wa3, tails, fidx, aidx, out,
          row_v, idx_v, out_v, sidx_v, idx1_v, sem_r, sem_i, sem_o):
    c = lax.axis_index("c")
    s = lax.axis_index("s")
    z16 = lax.iota(jnp.int32, _LANES) * 0

    out_cp = None
    for rep in range(_REPS):
      d = s * _REPS + rep
      sdim = d // 8
      r = d % 8
      idx1_v[pl.ds(0, _LANES)] = r + z16

      @pl.when(c == 0)
      def _():
        for off, sz in _PCH:
          pltpu.async_copy(
              wf3.at[sdim, :, pl.ds(off, sz)].at[idx1_v.at[pl.ds(0, 1)]],
              row_v.at[:, pl.ds(off, sz)], sem_r)

      @pl.when(c == 1)
      def _():
        for off, sz in _PCH:
          pltpu.async_copy(
              wa3.at[sdim, :, pl.ds(off, sz)].at[idx1_v.at[pl.ds(0, 1)]],
              row_v.at[:, pl.ds(off, sz)], sem_r)

      pltpu.async_copy(
          tails.at[c, sdim].at[idx1_v.at[pl.ds(0, 1)]],
          row_v.at[:, pl.ds(_MAIN, 128)], sem_r)

      idx_cps = []
      for ch in range(2):
        @pl.when(c == 0)
        def _():
          pltpu.async_copy(
              fidx.at[pl.ds(ch * _ICH, _ICH)], idx_v.at[ch], sem_i)

        @pl.when(c == 1)
        def _():
          pltpu.async_copy(
              aidx.at[pl.ds(ch * _ICH, _ICH)], idx_v.at[ch], sem_i)

      base = (c * 4 + sdim) * (8 * _TPD) + r
      def sbody(k, _):
        t16 = lax.iota(jnp.int32, _LANES) + k * _LANES
        sidx_v[rep, pl.ds(k * _LANES, _LANES)] = base + 8 * t16
        return 0
      lax.fori_loop(0, _TPD // _LANES, sbody, 0)

      for off, sz in _PCH:
        pltpu.make_async_copy(
            wf3.at[0, :, pl.ds(off, sz)].at[idx1_v.at[pl.ds(0, 1)]],
            row_v.at[:, pl.ds(off, sz)], sem_r).wait()
      pltpu.make_async_copy(
          tails.at[0, 0].at[idx1_v.at[pl.ds(0, 1)]],
          row_v.at[:, pl.ds(_MAIN, 128)], sem_r).wait()

      if out_cp is not None:
        out_cp.wait()

      for ch in range(_NICH):
        buf = ch % 2
        pltpu.make_async_copy(
            fidx.at[pl.ds(0, _ICH)], idx_v.at[buf], sem_i).wait()

        def body(i, _):
          base_i = i * (_UNROLL * _LANES)
          for u in range(_UNROLL):
            off = base_i + u * _LANES
            iv = idx_v[buf, pl.ds(off, _LANES)]
            vals = plsc.load_gather(row_v, [z16, iv])
            orow = (ch * _ICH + off) // 128
            ocol = off % 128
            out_v[orow, pl.ds(ocol, _LANES)] = vals
          return 0

        lax.fori_loop(0, _ICH // (_UNROLL * _LANES), body, 0)

        if ch + 2 < _NICH:
          nxt = ch + 2

          @pl.when(c == 0)
          def _():
            pltpu.async_copy(
                fidx.at[pl.ds(nxt * _ICH, _ICH)], idx_v.at[buf], sem_i)

          @pl.when(c == 1)
          def _():
            pltpu.async_copy(
                aidx.at[pl.ds(nxt * _ICH, _ICH)], idx_v.at[buf], sem_i)

      out_cp = pltpu.async_copy(out_v, out.at[sidx_v.at[rep]], sem_o)
    out_cp.wait()

  return emb


_emb = _build()


def kernel(user_fea, W_fans, W_avgrating):
  fans_idx = user_fea[:, 0].astype(jnp.int32)
  avg_idx = user_fea[:, 1].astype(jnp.int32)
  wf3 = W_fans.T.reshape(4, 8, _ROWS)
  wa3 = W_avgrating.T.reshape(4, 8, _ROWS)
  tails = jnp.stack((W_fans.T[:, _MAIN:], W_avgrating.T[:, _MAIN:]))
  tails = jnp.pad(tails, ((0, 0), (0, 0), (0, 128 - _TAIL)))
  tails = tails.reshape(2, 4, 8, 128)
  out4 = _emb(wf3, wa3, tails, fans_idx, avg_idx)
  out_t = (out4.reshape(8, 128, 8, 128)
           .transpose(0, 2, 1, 3)
           .reshape(2 * _DIM, _BATCH))
  return out_t.T


# speedup vs baseline: 1.0131x; 1.0012x over previous
import functools

import jax
import jax.numpy as jnp
from jax import lax
from jax.experimental import pallas as pl
from jax.experimental.pallas import tpu as pltpu
from jax.experimental.pallas import tpu_sc as plsc

_BATCH = 16384
_DIM = 32
_ROWS = 100000
_MAIN = 99968
_TAIL = _ROWS - _MAIN
_REPS = 2
_ICH = 4096
_PCH = ((0, 25600), (25600, 25600), (51200, 25600), (76800, 23168))
_NICH = _BATCH // _ICH
_UNROLL = 16
_LANES = 16
_TPD = _BATCH // 128


def _build():
  mesh = plsc.VectorSubcoreMesh(core_axis_name="c", subcore_axis_name="s")

  @functools.partial(
      pl.kernel,
      mesh=mesh,
      out_type=jax.ShapeDtypeStruct((2 * _DIM * _BATCH // 128, 128),
                                    jnp.float32),
      scratch_types=[
          pltpu.VMEM((1, _MAIN + 128), jnp.float32),
          pltpu.VMEM((2, _ICH), jnp.int32),
          pltpu.VMEM((_TPD, 128), jnp.float32),
          pltpu.VMEM((2, _TPD), jnp.int32),
          pltpu.VMEM((_LANES,), jnp.int32),
          pltpu.SemaphoreType.DMA,
          pltpu.SemaphoreType.DMA,
          pltpu.SemaphoreType.DMA,
      ],
      compiler_params=pltpu.CompilerParams(
          needs_layout_passes=False, disable_bounds_checks=True),
  )
  def emb(wf3, wa3, tails, fidx, aidx, out,
          row_v, idx_v, out_v, sidx_v, idx1_v, sem_r, sem_i, sem_o):
    c = lax.axis_index("c")
    s = lax.axis_index("s")
    z16 = lax.iota(jnp.int32, _LANES) * 0

    out_cp = None
    for rep in range(_REPS):
      d = s * _REPS + rep
      sdim = d // 8
      r = d % 8
      idx1_v[pl.ds(0, _LANES)] = r + z16

      @pl.when(c == 0)
      def _():
        for off, sz in _PCH:
          pltpu.async_copy(
              wf3.at[sdim, :, pl.ds(off, sz)].at[idx1_v.at[pl.ds(0, 1)]],
              row_v.at[:, pl.ds(off, sz)], sem_r)

      @pl.when(c == 1)
      def _():
        for off, sz in _PCH:
          pltpu.async_copy(
              wa3.at[sdim, :, pl.ds(off, sz)].at[idx1_v.at[pl.ds(0, 1)]],
              row_v.at[:, pl.ds(off, sz)], sem_r)

      pltpu.async_copy(
          tails.at[c, sdim].at[idx1_v.at[pl.ds(0, 1)]],
          row_v.at[:, pl.ds(_MAIN, 128)], sem_r)

      idx_cps = []
      for ch in range(2):
        @pl.when(c == 0)
        def _():
          pltpu.async_copy(
              fidx.at[pl.ds(ch * _ICH, _ICH)], idx_v.at[ch], sem_i)

        @pl.when(c == 1)
        def _():
          pltpu.async_copy(
              aidx.at[pl.ds(ch * _ICH, _ICH)], idx_v.at[ch], sem_i)

      base = (c * 4 + sdim) * (8 * _TPD) + r
      def sbody(k, _):
        t16 = lax.iota(jnp.int32, _LANES) + k * _LANES
        sidx_v[rep, pl.ds(k * _LANES, _LANES)] = base + 8 * t16
        return 0
      lax.fori_loop(0, _TPD // _LANES, sbody, 0)

      for off, sz in _PCH:
        pltpu.make_async_copy(
            wf3.at[0, :, pl.ds(off, sz)].at[idx1_v.at[pl.ds(0, 1)]],
            row_v.at[:, pl.ds(off, sz)], sem_r).wait()
      pltpu.make_async_copy(
          tails.at[0, 0].at[idx1_v.at[pl.ds(0, 1)]],
          row_v.at[:, pl.ds(_MAIN, 128)], sem_r).wait()

      if out_cp is not None:
        out_cp.wait()

      for ch in range(_NICH):
        buf = ch % 2
        pltpu.make_async_copy(
            fidx.at[pl.ds(0, _ICH)], idx_v.at[buf], sem_i).wait()

        def body(i, _):
          base_i = i * (_UNROLL * _LANES)
          for u in range(_UNROLL):
            off = base_i + u * _LANES
            iv = idx_v[buf, pl.ds(off, _LANES)]
            vals = plsc.load_gather(row_v, [z16, iv])
            orow = (ch * _ICH + off) // 128
            ocol = off % 128
            out_v[orow, pl.ds(ocol, _LANES)] = vals
          return 0

        lax.fori_loop(0, _ICH // (_UNROLL * _LANES), body, 0)

        if ch + 2 < _NICH:
          nxt = ch + 2

          @pl.when(c == 0)
          def _():
            pltpu.async_copy(
                fidx.at[pl.ds(nxt * _ICH, _ICH)], idx_v.at[buf], sem_i)

          @pl.when(c == 1)
          def _():
            pltpu.async_copy(
                aidx.at[pl.ds(nxt * _ICH, _ICH)], idx_v.at[buf], sem_i)

      out_cp = pltpu.async_copy(out_v, out.at[sidx_v.at[rep]], sem_o)
    out_cp.wait()

  return emb


_emb = _build()


def kernel(user_fea, W_fans, W_avgrating):
  fans_idx = user_fea[:, 0].astype(jnp.int32)
  avg_idx = user_fea[:, 1].astype(jnp.int32)
  wf3 = W_fans.T.reshape(4, 8, _ROWS)
  wa3 = W_avgrating.T.reshape(4, 8, _ROWS)
  tails = jnp.stack((W_fans.T[:, _MAIN:], W_avgrating.T[:, _MAIN:]))
  tails = jnp.pad(tails, ((0, 0), (0, 0), (0, 128 - _TAIL)))
  tails = tails.reshape(2, 4, 8, 128)
  out4 = _emb(wf3, wa3, tails, fans_idx, avg_idx)
  out_t = (out4.reshape(8, 128, 8, 128)
           .transpose(0, 2, 1, 3)
           .reshape(2 * _DIM, _BATCH))
  return out_t.T
